# baseline (speedup 1.0000x reference)
import functools

import jax
import jax.numpy as jnp
from jax import lax
from jax.experimental import pallas as pl
from jax.experimental.pallas import tpu as pltpu
from jax.experimental.pallas import tpu_sc as plsc

F32 = jnp.float32
BF16 = jnp.bfloat16
ET = 331776
TCB = 4096
GRID = ET // TCB
NW = 32
B2 = 96
I2 = ET // NW // B2
B6 = 128
I6 = ET // 16 // B6
NDUMMY = 240


def _k1_body(x_ref, lw_ref, sw_ref, dw_ref, xl_ref, as_ref, ad_ref):
    xb = x_ref[...]
    xl_ref[...] = jnp.dot(xb, lw_ref[...], preferred_element_type=F32)
    as_ref[...] = jnp.dot(xb, sw_ref[...], preferred_element_type=F32)
    ad_ref[...] = jnp.dot(xb, dw_ref[...], preferred_element_type=F32)


def _sc_gather(tsrc, tdst, src3, dst3):
    ws = tsrc.shape[1]
    wd = tdst.shape[1]
    mesh = plsc.VectorSubcoreMesh(core_axis_name="c", subcore_axis_name="s")
    per_w = ET // NW

    @functools.partial(
        pl.kernel, mesh=mesh,
        out_type=[
            jax.ShapeDtypeStruct((ET, ws), jnp.int32),
            jax.ShapeDtypeStruct((ET, wd), jnp.int32),
        ],
        scratch_types=[
            pltpu.VMEM((I2, B2), jnp.int32),
            pltpu.VMEM((I2, B2), jnp.int32),
            pltpu.VMEM((B2, ws), jnp.int32),
            pltpu.VMEM((B2, ws), jnp.int32),
            pltpu.VMEM((B2, wd), jnp.int32),
            pltpu.VMEM((B2, wd), jnp.int32),
            pltpu.SemaphoreType.DMA,
            pltpu.SemaphoreType.DMA,
            pltpu.SemaphoreType.DMA,
            pltpu.SemaphoreType.DMA,
        ],
    )
    def k2(tsrc_h, tdst_h, src_h, dst_h, gs_h, gd_h,
           ixs, ixd, bsA, bsB, bdA, bdB, sgA, sgB, swA, swB):
        wid = lax.axis_index("s") * 2 + lax.axis_index("c")
        base = wid * per_w
        pltpu.sync_copy(src_h.at[wid], ixs)
        pltpu.sync_copy(dst_h.at[wid], ixd)

        def g_start(i, bs, bd, sg):
            pltpu.async_copy(tsrc_h.at[ixs.at[i]], bs, sg)
            pltpu.async_copy(tdst_h.at[ixd.at[i]], bd, sg)

        def g_wait(i, bs, bd, sg):
            pltpu.make_async_copy(tsrc_h.at[ixs.at[i]], bs, sg).wait()
            pltpu.make_async_copy(tdst_h.at[ixd.at[i]], bd, sg).wait()

        def w_start(i, bs, bd, sw):
            off = base + i * B2
            pltpu.async_copy(bs, gs_h.at[pl.ds(off, B2)], sw)
            pltpu.async_copy(bd, gd_h.at[pl.ds(off, B2)], sw)

        def w_wait(i, bs, bd, sw):
            off = base + i * B2
            pltpu.make_async_copy(bs, gs_h.at[pl.ds(off, B2)], sw).wait()
            pltpu.make_async_copy(bd, gd_h.at[pl.ds(off, B2)], sw).wait()

        g_start(0, bsA, bdA, sgA)

        @pl.loop(0, I2 // 2)
        def _(p):
            i0 = 2 * p
            i1 = i0 + 1
            g_wait(i0, bsA, bdA, sgA)
            w_start(i0, bsA, bdA, swA)

            @pl.when(p > 0)
            def _():
                w_wait(i0 - 1, bsB, bdB, swB)

            g_start(i1, bsB, bdB, sgB)
            w_wait(i0, bsA, bdA, swA)

            @pl.when(p < I2 // 2 - 1)
            def _():
                g_start(i0 + 2, bsA, bdA, sgA)

            g_wait(i1, bsB, bdB, sgB)
            w_start(i1, bsB, bdB, swB)

        w_wait(I2 - 1, bsB, bdB, swB)

    return k2(tsrc, tdst, src3, dst3)


def _hi(x):
    return lax.bitcast_convert_type(x & jnp.int32(-65536), F32)


def _lo(x):
    return lax.bitcast_convert_type(x << 16, F32)


def _pkhi(x):
    xi = lax.bitcast_convert_type(x, jnp.int32)
    r = xi + jnp.int32(0x7FFF) + (lax.shift_right_logical(xi, 16) & 1)
    return r & jnp.int32(-65536)


def _pk2(a, b):
    return _pkhi(a) | lax.shift_right_logical(_pkhi(b), 16)


def _k3_body(n, gs0_ref, gd0_ref, s2_ref, d2_ref, w1_ref, b1_ref,
             out_ref, acc_ref, cnt_ref):
    i = pl.program_id(0)

    @pl.when(i == 0)
    def _():
        acc_ref[...] = jnp.zeros_like(acc_ref)
        cnt_ref[0] = 0.0

    pdiff = _lo(gd0_ref[...]) - _lo(gs0_ref[...])
    h1 = jnp.dot(pdiff.astype(BF16), w1_ref[...],
                 preferred_element_type=F32) + b1_ref[...]
    acc_ref[0:1, :] += jnp.sum(h1, axis=0, keepdims=True)
    acc_ref[1:2, :] += jnp.sum(h1 * h1, axis=0, keepdims=True)
    cnt_ref[0] += jnp.sum((s2_ref[...] != d2_ref[...]).astype(F32))

    @pl.when(i == pl.num_programs(0) - 1)
    def _():
        dh = acc_ref.shape[1]
        cnt = cnt_ref[0] + float(n)
        out_ref[...] = jnp.concatenate(
            [acc_ref[0:2, :], jnp.full((1, dh), cnt, F32),
             jnp.zeros((5, dh), F32)], axis=0)


def _k4_body(gs0_ref, gd0_ref, st_ref, w1_ref, b1_ref,
             g_ref, be_ref, w2_ref, b2_ref, aw1_ref, ab1_ref,
             p1_ref, out_ref, acc_ref):
    i = pl.program_id(0)

    @pl.when(i == 0)
    def _():
        acc_ref[...] = jnp.zeros_like(acc_ref)

    cntv = st_ref[2:3, :]
    ndv = float(ET) - cntv
    b1 = b1_ref[...]
    muv = (st_ref[0:1, :] - ndv * b1) / cntv
    msq = (st_ref[1:2, :] - ndv * b1 * b1) / cntv
    varv = msq - muv * muv
    s1 = g_ref[...] * jax.lax.rsqrt(varv + 1e-5)
    sh = be_ref[...] - muv * s1

    gs0 = gs0_ref[...]
    gd0 = gd0_ref[...]
    pdiff = _lo(gd0) - _lo(gs0)
    h1 = jnp.dot(pdiff.astype(BF16), w1_ref[...],
                 preferred_element_type=F32) + b1
    hbn = jnp.maximum(h1 * s1 + sh, 0.0)
    delta = jnp.dot(hbn.astype(BF16), w2_ref[...],
                    preferred_element_type=F32) + b2_ref[...]
    adf = _hi(gd0) - _hi(gs0)
    p1_ref[...] = _pk2(delta, adf)
    h1a = jnp.dot((adf + delta).astype(BF16), aw1_ref[...],
                  preferred_element_type=F32) + ab1_ref[...]
    acc_ref[0:1, :] += jnp.sum(h1a, axis=0, keepdims=True)
    acc_ref[1:2, :] += jnp.sum(h1a * h1a, axis=0, keepdims=True)

    @pl.when(i == pl.num_programs(0) - 1)
    def _():
        dh = acc_ref.shape[1]
        hbn0 = jnp.maximum(b1 * s1 + sh, 0.0)
        delta0 = jnp.dot(hbn0.astype(BF16), w2_ref[...],
                         preferred_element_type=F32) + b2_ref[...]
        h1a0 = jnp.dot(delta0.astype(BF16), aw1_ref[...],
                       preferred_element_type=F32) + ab1_ref[...]
        amu = (acc_ref[0:1, :] - ndv * h1a0) / cntv
        asq = (acc_ref[1:2, :] - ndv * h1a0 * h1a0) / cntv
        avar = asq - amu * amu
        out_ref[...] = jnp.concatenate(
            [amu, avar, jnp.zeros((6, dh), F32)], axis=0)


def _k5_body(p1_ref, ap_ref, ag_ref, abe_ref, aw1_ref, ab1_ref,
             aw2_ref, ab2_ref, u_ref, mx_ref, macc_ref):
    i = pl.program_id(0)

    @pl.when(i == 0)
    def _():
        macc_ref[...] = jnp.full_like(macc_ref, -3e38)

    s2 = ag_ref[...] * jax.lax.rsqrt(ap_ref[1:2, :] + 1e-5)
    sh2 = abe_ref[...] - ap_ref[0:1, :] * s2
    p1 = p1_ref[...]
    h1a = jnp.dot((_lo(p1) + _hi(p1)).astype(BF16), aw1_ref[...],
                  preferred_element_type=F32) + ab1_ref[...]
    ha = jnp.maximum(h1a * s2 + sh2, 0.0)
    u = jnp.dot(ha.astype(BF16), aw2_ref[...],
                preferred_element_type=F32) + ab2_ref[...]
    u_ref[...] = u
    macc_ref[0:1, :] = jnp.maximum(macc_ref[0:1, :],
                                   jnp.max(u, axis=0, keepdims=True))

    @pl.when(i == pl.num_programs(0) - 1)
    def _():
        mx_ref[...] = jnp.broadcast_to(macc_ref[0:1, :], mx_ref.shape)


def _k5b_body(u_ref, p1_ref, gs1_ref, mx_ref, e_ref, m_ref):
    e = jnp.exp(u_ref[...] - mx_ref[0:1, :])
    e_ref[...] = e
    m_ref[...] = e * (_hi(gs1_ref[...]) + _hi(p1_ref[...]))


def _sc_scatter(m, e, dst3, nacc):
    d = m.shape[1]
    mesh = plsc.VectorSubcoreMesh(core_axis_name="c", subcore_axis_name="s")
    per_sub = ET // 16
    rps = nacc // 16

    @functools.partial(
        pl.kernel, mesh=mesh,
        out_type=[
            jax.ShapeDtypeStruct((nacc, d), F32),
            jax.ShapeDtypeStruct((nacc, d), F32),
        ],
        scratch_types=[
            pltpu.VMEM((2, B6), jnp.int32),
            pltpu.VMEM((B6, d), F32),
            pltpu.VMEM((B6, d), F32),
            pltpu.VMEM_SHARED((nacc, d), F32),
            pltpu.SemaphoreType.DMA,
            pltpu.SemaphoreType.DMA,
        ],
    )
    def k6(m_h, e_h, dst_h, accm_h, acce_h,
           idx, rowA, rowB, spacc, sA, sB):
        c = lax.axis_index("c")
        sid = lax.axis_index("s")

        @pl.loop(0, B6)
        def _(r):
            @pl.loop(0, d // 16)
            def _(q):
                rowA[r, pl.ds(q * 16, 16)] = jnp.zeros((16,), F32)

        @pl.loop(0, rps // B6)
        def _(k):
            pltpu.sync_copy(rowA, spacc.at[pl.ds(sid * rps + k * B6, B6)])

        plsc.subcore_barrier()

        def run(arr_h, out_h):
            def l_start(i, buf, sem):
                off = sid * per_sub + i * B6
                pltpu.async_copy(arr_h.at[pl.ds(off, B6)], buf, sem)

            def l_wait(i, buf, sem):
                off = sid * per_sub + i * B6
                pltpu.make_async_copy(arr_h.at[pl.ds(off, B6)], buf,
                                      sem).wait()

            l_start(0, rowA, sA)

            @pl.loop(0, I6 // 2)
            def _(p):
                i0 = 2 * p
                i1 = i0 + 1
                pltpu.sync_copy(dst_h.at[sid, pl.ds(i0, 2)], idx)
                l_wait(i0, rowA, sA)
                l_start(i1, rowB, sB)
                pltpu.sync_copy(rowA, spacc.at[idx.at[0]], add=True)
                l_wait(i1, rowB, sB)

                @pl.when(p < I6 // 2 - 1)
                def _():
                    l_start(i0 + 2, rowA, sA)

                pltpu.sync_copy(rowB, spacc.at[idx.at[1]], add=True)

            plsc.subcore_barrier()
            pltpu.sync_copy(spacc.at[pl.ds(sid * rps, rps)],
                            out_h.at[pl.ds(sid * rps, rps)])

        @pl.when(c == 0)
        def _():
            run(m_h, accm_h)

        @pl.when(c == 1)
        def _():
            run(e_h, acce_h)

    return k6(m, e, dst3)


def _k7_body(num_ref, den_ref, out_ref):
    out_ref[...] = num_ref[...] / (den_ref[...] + 1e-16)


def _espec(w, col=0):
    return pl.BlockSpec((TCB, w), lambda i, c=col: (i, c))


def _cspec(shape):
    return pl.BlockSpec(shape, lambda i: (0, 0))


def kernel(x, pos, edge_index, lin_W, src_W, dst_W, pos_W1, pos_b1, pos_g,
           pos_be, pos_W2, pos_b2, attn_W1, attn_b1, attn_g, attn_be,
           attn_W2, attn_b2):
    n, d = x.shape
    e = edge_index.shape[1]
    dh = pos_W1.shape[1]
    pad = ET - e - n

    src0 = edge_index[0].astype(jnp.int32)
    dst0 = edge_index[1].astype(jnp.int32)
    loops = jnp.arange(n, dtype=jnp.int32)
    padi = jnp.arange(pad, dtype=jnp.int32) % n
    srcp = jnp.concatenate([src0, loops, padi])
    dstp = jnp.concatenate([dst0, loops, padi])
    nacc = n + NDUMMY
    maskv = jnp.concatenate([src0 != dst0, jnp.ones((n,), bool),
                             jnp.zeros((pad,), bool)])
    dmy = n + (jnp.arange(ET, dtype=jnp.int32) % NDUMMY)
    dsts = jnp.where(maskv, dstp, dmy)
    posp = jnp.pad(pos.astype(F32), ((0, 0), (0, 128 - pos.shape[1])))
    w1p = jnp.pad(pos_W1, ((0, 128 - pos_W1.shape[0]), (0, 0))).astype(BF16)
    pw2b = pos_W2.astype(BF16)
    aw1b = attn_W1.astype(BF16)
    aw2b = attn_W2.astype(BF16)
    pb1 = pos_b1.reshape(1, dh)
    pg = pos_g.reshape(1, dh)
    pbe = pos_be.reshape(1, dh)
    pb2 = pos_b2.reshape(1, d)
    ab1 = attn_b1.reshape(1, dh)
    ag = attn_g.reshape(1, dh)
    abe = attn_be.reshape(1, dh)
    ab2 = attn_b2.reshape(1, d)
    s2d = srcp.reshape(ET // 512, 512)
    d2d = dstp.reshape(ET // 512, 512)

    xl, aS, aD = pl.pallas_call(
        _k1_body,
        out_shape=[jax.ShapeDtypeStruct((n, d), F32)] * 3,
    )(x, lin_W, src_W, dst_W)

    def pack2(A, B):
        au = lax.bitcast_convert_type(A.astype(BF16), jnp.uint16)
        bu = lax.bitcast_convert_type(B.astype(BF16), jnp.uint16)
        w = (au.astype(jnp.uint32) << 16) | bu.astype(jnp.uint32)
        return lax.bitcast_convert_type(w, jnp.int32)

    tsrc = jnp.concatenate(
        [pack2(aS, posp), pack2(xl, jnp.zeros((n, d), F32))], axis=1)
    tdst = pack2(aD, posp)
    src3 = srcp.reshape(NW, I2, B2)
    dst3 = dstp.reshape(NW, I2, B2)
    gs, gd = _sc_gather(tsrc, tdst, src3, dst3)

    st = pl.pallas_call(
        functools.partial(_k3_body, n),
        grid=(GRID,),
        in_specs=[_espec(d, 0), _espec(d, 0),
                  pl.BlockSpec((8, 512), lambda i: (i, 0)),
                  pl.BlockSpec((8, 512), lambda i: (i, 0)),
                  _cspec((128, dh)), _cspec((1, dh))],
        out_specs=_cspec((8, dh)),
        out_shape=jax.ShapeDtypeStruct((8, dh), F32),
        scratch_shapes=[pltpu.VMEM((8, dh), F32), pltpu.SMEM((1,), F32)],
    )(gs, gd, s2d, d2d, w1p, pb1)

    p1, ap = pl.pallas_call(
        _k4_body,
        grid=(GRID,),
        in_specs=[_espec(d, 0), _espec(d, 0),
                  _cspec((8, dh)), _cspec((128, dh)), _cspec((1, dh)),
                  _cspec((1, dh)), _cspec((1, dh)), _cspec((dh, d)),
                  _cspec((1, d)), _cspec((d, dh)), _cspec((1, dh))],
        out_specs=[_espec(d), _cspec((8, dh))],
        out_shape=[jax.ShapeDtypeStruct((ET, d), jnp.int32),
                   jax.ShapeDtypeStruct((8, dh), F32)],
        scratch_shapes=[pltpu.VMEM((8, dh), F32)],
    )(gs, gd, st, w1p, pb1, pg, pbe, pw2b, pb2, aw1b, ab1)

    u, mx = pl.pallas_call(
        _k5_body,
        grid=(GRID,),
        in_specs=[_espec(d), _cspec((8, dh)), _cspec((1, dh)),
                  _cspec((1, dh)), _cspec((d, dh)), _cspec((1, dh)),
                  _cspec((dh, d)), _cspec((1, d))],
        out_specs=[_espec(d), _cspec((8, d))],
        out_shape=[jax.ShapeDtypeStruct((ET, d), F32),
                   jax.ShapeDtypeStruct((8, d), F32)],
        scratch_shapes=[pltpu.VMEM((8, d), F32)],
    )(p1, ap, ag, abe, aw1b, ab1, aw2b, ab2)

    ev, mv = pl.pallas_call(
        _k5b_body,
        grid=(GRID,),
        in_specs=[_espec(d), _espec(d), _espec(d, 1), _cspec((8, d))],
        out_specs=[_espec(d), _espec(d)],
        out_shape=[jax.ShapeDtypeStruct((ET, d), F32),
                   jax.ShapeDtypeStruct((ET, d), F32)],
    )(u, p1, gs, mx)

    dst3s = dsts.reshape(16, I6, B6)
    accm, acce = _sc_scatter(mv, ev, dst3s, nacc)

    out = pl.pallas_call(
        _k7_body,
        out_shape=jax.ShapeDtypeStruct((n, d), F32),
    )(accm[:n], acce[:n])
    return out


# speedup vs baseline: 1.0155x; 1.0155x over previous
import functools

import jax
import jax.numpy as jnp
from jax import lax
from jax.experimental import pallas as pl
from jax.experimental.pallas import tpu as pltpu
from jax.experimental.pallas import tpu_sc as plsc

F32 = jnp.float32
BF16 = jnp.bfloat16
ET = 331776
TCB = 4096
GRID = ET // TCB
NW = 32
B2 = 96
I2 = ET // NW // B2
B6 = 128
I6 = ET // 16 // B6
NDUMMY = 240


def _k1_body(x_ref, lw_ref, sw_ref, dw_ref, xl_ref, as_ref, ad_ref):
    xb = x_ref[...]
    xl_ref[...] = jnp.dot(xb, lw_ref[...], preferred_element_type=F32)
    as_ref[...] = jnp.dot(xb, sw_ref[...], preferred_element_type=F32)
    ad_ref[...] = jnp.dot(xb, dw_ref[...], preferred_element_type=F32)


def _sc_gather(tsrc, tdst, src3, dst3):
    ws = tsrc.shape[1]
    wd = tdst.shape[1]
    mesh = plsc.VectorSubcoreMesh(core_axis_name="c", subcore_axis_name="s")
    per_w = ET // NW

    @functools.partial(
        pl.kernel, mesh=mesh,
        out_type=[
            jax.ShapeDtypeStruct((ET, ws), jnp.int32),
            jax.ShapeDtypeStruct((ET, wd), jnp.int32),
        ],
        scratch_types=[
            pltpu.VMEM((I2, B2), jnp.int32),
            pltpu.VMEM((I2, B2), jnp.int32),
            pltpu.VMEM((B2, ws), jnp.int32),
            pltpu.VMEM((B2, ws), jnp.int32),
            pltpu.VMEM((B2, wd), jnp.int32),
            pltpu.VMEM((B2, wd), jnp.int32),
            pltpu.SemaphoreType.DMA,
            pltpu.SemaphoreType.DMA,
            pltpu.SemaphoreType.DMA,
            pltpu.SemaphoreType.DMA,
        ],
    )
    def k2(tsrc_h, tdst_h, src_h, dst_h, gs_h, gd_h,
           ixs, ixd, bsA, bsB, bdA, bdB, sgA, sgB, swA, swB):
        wid = lax.axis_index("s") * 2 + lax.axis_index("c")
        base = wid * per_w
        pltpu.sync_copy(src_h.at[wid], ixs)
        pltpu.sync_copy(dst_h.at[wid], ixd)

        def g_start(i, bs, bd, sg):
            pltpu.async_copy(tsrc_h.at[ixs.at[i]], bs, sg)
            pltpu.async_copy(tdst_h.at[ixd.at[i]], bd, sg)

        def g_wait(i, bs, bd, sg):
            pltpu.make_async_copy(tsrc_h.at[ixs.at[i]], bs, sg).wait()
            pltpu.make_async_copy(tdst_h.at[ixd.at[i]], bd, sg).wait()

        def w_start(i, bs, bd, sw):
            off = base + i * B2
            pltpu.async_copy(bs, gs_h.at[pl.ds(off, B2)], sw)
            pltpu.async_copy(bd, gd_h.at[pl.ds(off, B2)], sw)

        def w_wait(i, bs, bd, sw):
            off = base + i * B2
            pltpu.make_async_copy(bs, gs_h.at[pl.ds(off, B2)], sw).wait()
            pltpu.make_async_copy(bd, gd_h.at[pl.ds(off, B2)], sw).wait()

        g_start(0, bsA, bdA, sgA)

        @pl.loop(0, I2 // 2)
        def _(p):
            i0 = 2 * p
            i1 = i0 + 1
            g_wait(i0, bsA, bdA, sgA)
            w_start(i0, bsA, bdA, swA)

            @pl.when(p > 0)
            def _():
                w_wait(i0 - 1, bsB, bdB, swB)

            g_start(i1, bsB, bdB, sgB)
            w_wait(i0, bsA, bdA, swA)

            @pl.when(p < I2 // 2 - 1)
            def _():
                g_start(i0 + 2, bsA, bdA, sgA)

            g_wait(i1, bsB, bdB, sgB)
            w_start(i1, bsB, bdB, swB)

        w_wait(I2 - 1, bsB, bdB, swB)

    return k2(tsrc, tdst, src3, dst3)


def _hi(x):
    return lax.bitcast_convert_type(x & jnp.int32(-65536), F32)


def _lo(x):
    return lax.bitcast_convert_type(x << 16, F32)


def _pkhi(x):
    xi = lax.bitcast_convert_type(x, jnp.int32)
    r = xi + jnp.int32(0x7FFF) + (lax.shift_right_logical(xi, 16) & 1)
    return r & jnp.int32(-65536)


def _pk2(a, b):
    return _pkhi(a) | lax.shift_right_logical(_pkhi(b), 16)


def _k3_body(n, gs0_ref, gd0_ref, s2_ref, d2_ref, w1_ref, b1_ref,
             out_ref, acc_ref, cnt_ref):
    i = pl.program_id(0)

    @pl.when(i == 0)
    def _():
        acc_ref[...] = jnp.zeros_like(acc_ref)
        cnt_ref[0] = 0.0

    pdiff = _lo(gd0_ref[...]) - _lo(gs0_ref[...])
    h1 = jnp.dot(pdiff.astype(BF16), w1_ref[...],
                 preferred_element_type=F32) + b1_ref[...]
    acc_ref[0:1, :] += jnp.sum(h1, axis=0, keepdims=True)
    acc_ref[1:2, :] += jnp.sum(h1 * h1, axis=0, keepdims=True)
    cnt_ref[0] += jnp.sum((s2_ref[...] != d2_ref[...]).astype(F32))

    @pl.when(i == pl.num_programs(0) - 1)
    def _():
        dh = acc_ref.shape[1]
        cnt = cnt_ref[0] + float(n)
        out_ref[...] = jnp.concatenate(
            [acc_ref[0:2, :], jnp.full((1, dh), cnt, F32),
             jnp.zeros((5, dh), F32)], axis=0)


def _k4_body(gs0_ref, gd0_ref, st_ref, w1_ref, b1_ref,
             g_ref, be_ref, w2_ref, b2_ref, aw1_ref, ab1_ref,
             p1_ref, out_ref, acc_ref):
    i = pl.program_id(0)

    @pl.when(i == 0)
    def _():
        acc_ref[...] = jnp.zeros_like(acc_ref)

    cntv = st_ref[2:3, :]
    ndv = float(ET) - cntv
    b1 = b1_ref[...]
    muv = (st_ref[0:1, :] - ndv * b1) / cntv
    msq = (st_ref[1:2, :] - ndv * b1 * b1) / cntv
    varv = msq - muv * muv
    s1 = g_ref[...] * jax.lax.rsqrt(varv + 1e-5)
    sh = be_ref[...] - muv * s1

    gs0 = gs0_ref[...]
    gd0 = gd0_ref[...]
    pdiff = _lo(gd0) - _lo(gs0)
    h1 = jnp.dot(pdiff.astype(BF16), w1_ref[...],
                 preferred_element_type=F32) + b1
    hbn = jnp.maximum(h1 * s1 + sh, 0.0)
    delta = jnp.dot(hbn.astype(BF16), w2_ref[...],
                    preferred_element_type=F32) + b2_ref[...]
    adf = _hi(gd0) - _hi(gs0)
    p1_ref[...] = _pk2(delta, adf)
    h1a = jnp.dot((adf + delta).astype(BF16), aw1_ref[...],
                  preferred_element_type=F32) + ab1_ref[...]
    acc_ref[0:1, :] += jnp.sum(h1a, axis=0, keepdims=True)
    acc_ref[1:2, :] += jnp.sum(h1a * h1a, axis=0, keepdims=True)

    @pl.when(i == pl.num_programs(0) - 1)
    def _():
        dh = acc_ref.shape[1]
        hbn0 = jnp.maximum(b1 * s1 + sh, 0.0)
        delta0 = jnp.dot(hbn0.astype(BF16), w2_ref[...],
                         preferred_element_type=F32) + b2_ref[...]
        h1a0 = jnp.dot(delta0.astype(BF16), aw1_ref[...],
                       preferred_element_type=F32) + ab1_ref[...]
        amu = (acc_ref[0:1, :] - ndv * h1a0) / cntv
        asq = (acc_ref[1:2, :] - ndv * h1a0 * h1a0) / cntv
        avar = asq - amu * amu
        out_ref[...] = jnp.concatenate(
            [amu, avar, jnp.zeros((6, dh), F32)], axis=0)


def _k5_body(p1_ref, ap_ref, ag_ref, abe_ref, aw1_ref, ab1_ref,
             aw2_ref, ab2_ref, u_ref, mx_ref, macc_ref):
    i = pl.program_id(0)

    @pl.when(i == 0)
    def _():
        macc_ref[...] = jnp.full_like(macc_ref, -3e38)

    s2 = ag_ref[...] * jax.lax.rsqrt(ap_ref[1:2, :] + 1e-5)
    sh2 = abe_ref[...] - ap_ref[0:1, :] * s2
    p1 = p1_ref[...]
    h1a = jnp.dot((_lo(p1) + _hi(p1)).astype(BF16), aw1_ref[...],
                  preferred_element_type=F32) + ab1_ref[...]
    ha = jnp.maximum(h1a * s2 + sh2, 0.0)
    u = jnp.dot(ha.astype(BF16), aw2_ref[...],
                preferred_element_type=F32) + ab2_ref[...]
    u_ref[...] = _pk2(u, _hi(p1))
    macc_ref[0:1, :] = jnp.maximum(macc_ref[0:1, :],
                                   jnp.max(u, axis=0, keepdims=True))

    @pl.when(i == pl.num_programs(0) - 1)
    def _():
        mx_ref[...] = jnp.broadcast_to(macc_ref[0:1, :], mx_ref.shape)


def _k5b_body(p2_ref, gs1_ref, mx_ref, e_ref, m_ref):
    p2 = p2_ref[...]
    e = jnp.exp(_hi(p2) - mx_ref[0:1, :])
    e_ref[...] = e
    m_ref[...] = e * (_hi(gs1_ref[...]) + _lo(p2))


def _sc_scatter(m, e, dst3, nacc):
    d = m.shape[1]
    mesh = plsc.VectorSubcoreMesh(core_axis_name="c", subcore_axis_name="s")
    per_sub = ET // 16
    rps = nacc // 16

    @functools.partial(
        pl.kernel, mesh=mesh,
        out_type=[
            jax.ShapeDtypeStruct((nacc, d), F32),
            jax.ShapeDtypeStruct((nacc, d), F32),
        ],
        scratch_types=[
            pltpu.VMEM((2, B6), jnp.int32),
            pltpu.VMEM((B6, d), F32),
            pltpu.VMEM((B6, d), F32),
            pltpu.VMEM_SHARED((nacc, d), F32),
            pltpu.SemaphoreType.DMA,
            pltpu.SemaphoreType.DMA,
        ],
    )
    def k6(m_h, e_h, dst_h, accm_h, acce_h,
           idx, rowA, rowB, spacc, sA, sB):
        c = lax.axis_index("c")
        sid = lax.axis_index("s")

        @pl.loop(0, B6)
        def _(r):
            @pl.loop(0, d // 16)
            def _(q):
                rowA[r, pl.ds(q * 16, 16)] = jnp.zeros((16,), F32)

        @pl.loop(0, rps // B6)
        def _(k):
            pltpu.sync_copy(rowA, spacc.at[pl.ds(sid * rps + k * B6, B6)])

        plsc.subcore_barrier()

        def run(arr_h, out_h):
            def l_start(i, buf, sem):
                off = sid * per_sub + i * B6
                pltpu.async_copy(arr_h.at[pl.ds(off, B6)], buf, sem)

            def l_wait(i, buf, sem):
                off = sid * per_sub + i * B6
                pltpu.make_async_copy(arr_h.at[pl.ds(off, B6)], buf,
                                      sem).wait()

            l_start(0, rowA, sA)

            @pl.loop(0, I6 // 2)
            def _(p):
                i0 = 2 * p
                i1 = i0 + 1
                pltpu.sync_copy(dst_h.at[sid, pl.ds(i0, 2)], idx)
                l_wait(i0, rowA, sA)
                l_start(i1, rowB, sB)
                pltpu.sync_copy(rowA, spacc.at[idx.at[0]], add=True)
                l_wait(i1, rowB, sB)

                @pl.when(p < I6 // 2 - 1)
                def _():
                    l_start(i0 + 2, rowA, sA)

                pltpu.sync_copy(rowB, spacc.at[idx.at[1]], add=True)

            plsc.subcore_barrier()
            pltpu.sync_copy(spacc.at[pl.ds(sid * rps, rps)],
                            out_h.at[pl.ds(sid * rps, rps)])

        @pl.when(c == 0)
        def _():
            run(m_h, accm_h)

        @pl.when(c == 1)
        def _():
            run(e_h, acce_h)

    return k6(m, e, dst3)


def _k7_body(num_ref, den_ref, out_ref):
    out_ref[...] = num_ref[...] / (den_ref[...] + 1e-16)


def _espec(w, col=0):
    return pl.BlockSpec((TCB, w), lambda i, c=col: (i, c))


def _cspec(shape):
    return pl.BlockSpec(shape, lambda i: (0, 0))


def kernel(x, pos, edge_index, lin_W, src_W, dst_W, pos_W1, pos_b1, pos_g,
           pos_be, pos_W2, pos_b2, attn_W1, attn_b1, attn_g, attn_be,
           attn_W2, attn_b2):
    n, d = x.shape
    e = edge_index.shape[1]
    dh = pos_W1.shape[1]
    pad = ET - e - n

    src0 = edge_index[0].astype(jnp.int32)
    dst0 = edge_index[1].astype(jnp.int32)
    loops = jnp.arange(n, dtype=jnp.int32)
    padi = jnp.arange(pad, dtype=jnp.int32) % n
    srcp = jnp.concatenate([src0, loops, padi])
    dstp = jnp.concatenate([dst0, loops, padi])
    nacc = n + NDUMMY
    maskv = jnp.concatenate([src0 != dst0, jnp.ones((n,), bool),
                             jnp.zeros((pad,), bool)])
    dmy = n + (jnp.arange(ET, dtype=jnp.int32) % NDUMMY)
    dsts = jnp.where(maskv, dstp, dmy)
    posp = jnp.pad(pos.astype(F32), ((0, 0), (0, 128 - pos.shape[1])))
    w1p = jnp.pad(pos_W1, ((0, 128 - pos_W1.shape[0]), (0, 0))).astype(BF16)
    pw2b = pos_W2.astype(BF16)
    aw1b = attn_W1.astype(BF16)
    aw2b = attn_W2.astype(BF16)
    pb1 = pos_b1.reshape(1, dh)
    pg = pos_g.reshape(1, dh)
    pbe = pos_be.reshape(1, dh)
    pb2 = pos_b2.reshape(1, d)
    ab1 = attn_b1.reshape(1, dh)
    ag = attn_g.reshape(1, dh)
    abe = attn_be.reshape(1, dh)
    ab2 = attn_b2.reshape(1, d)
    s2d = srcp.reshape(ET // 512, 512)
    d2d = dstp.reshape(ET // 512, 512)

    xl, aS, aD = pl.pallas_call(
        _k1_body,
        out_shape=[jax.ShapeDtypeStruct((n, d), F32)] * 3,
    )(x, lin_W, src_W, dst_W)

    def pack2(A, B):
        au = lax.bitcast_convert_type(A.astype(BF16), jnp.uint16)
        bu = lax.bitcast_convert_type(B.astype(BF16), jnp.uint16)
        w = (au.astype(jnp.uint32) << 16) | bu.astype(jnp.uint32)
        return lax.bitcast_convert_type(w, jnp.int32)

    tsrc = jnp.concatenate(
        [pack2(aS, posp), pack2(xl, jnp.zeros((n, d), F32))], axis=1)
    tdst = pack2(aD, posp)
    src3 = srcp.reshape(NW, I2, B2)
    dst3 = dstp.reshape(NW, I2, B2)
    gs, gd = _sc_gather(tsrc, tdst, src3, dst3)

    st = pl.pallas_call(
        functools.partial(_k3_body, n),
        grid=(GRID,),
        in_specs=[_espec(d, 0), _espec(d, 0),
                  pl.BlockSpec((8, 512), lambda i: (i, 0)),
                  pl.BlockSpec((8, 512), lambda i: (i, 0)),
                  _cspec((128, dh)), _cspec((1, dh))],
        out_specs=_cspec((8, dh)),
        out_shape=jax.ShapeDtypeStruct((8, dh), F32),
        scratch_shapes=[pltpu.VMEM((8, dh), F32), pltpu.SMEM((1,), F32)],
    )(gs, gd, s2d, d2d, w1p, pb1)

    p1, ap = pl.pallas_call(
        _k4_body,
        grid=(GRID,),
        in_specs=[_espec(d, 0), _espec(d, 0),
                  _cspec((8, dh)), _cspec((128, dh)), _cspec((1, dh)),
                  _cspec((1, dh)), _cspec((1, dh)), _cspec((dh, d)),
                  _cspec((1, d)), _cspec((d, dh)), _cspec((1, dh))],
        out_specs=[_espec(d), _cspec((8, dh))],
        out_shape=[jax.ShapeDtypeStruct((ET, d), jnp.int32),
                   jax.ShapeDtypeStruct((8, dh), F32)],
        scratch_shapes=[pltpu.VMEM((8, dh), F32)],
    )(gs, gd, st, w1p, pb1, pg, pbe, pw2b, pb2, aw1b, ab1)

    u, mx = pl.pallas_call(
        _k5_body,
        grid=(GRID,),
        in_specs=[_espec(d), _cspec((8, dh)), _cspec((1, dh)),
                  _cspec((1, dh)), _cspec((d, dh)), _cspec((1, dh)),
                  _cspec((dh, d)), _cspec((1, d))],
        out_specs=[_espec(d), _cspec((8, d))],
        out_shape=[jax.ShapeDtypeStruct((ET, d), jnp.int32),
                   jax.ShapeDtypeStruct((8, d), F32)],
        scratch_shapes=[pltpu.VMEM((8, d), F32)],
    )(p1, ap, ag, abe, aw1b, ab1, aw2b, ab2)

    ev, mv = pl.pallas_call(
        _k5b_body,
        grid=(GRID,),
        in_specs=[_espec(d), _espec(d, 1), _cspec((8, d))],
        out_specs=[_espec(d), _espec(d)],
        out_shape=[jax.ShapeDtypeStruct((ET, d), F32),
                   jax.ShapeDtypeStruct((ET, d), F32)],
    )(u, gs, mx)

    dst3s = dsts.reshape(16, I6, B6)
    accm, acce = _sc_scatter(mv, ev, dst3s, nacc)

    out = pl.pallas_call(
        _k7_body,
        out_shape=jax.ShapeDtypeStruct((n, d), F32),
    )(accm[:n], acce[:n])
    return out


# speedup vs baseline: 1.0316x; 1.0158x over previous
import functools

import jax
import jax.numpy as jnp
from jax import lax
from jax.experimental import pallas as pl
from jax.experimental.pallas import tpu as pltpu
from jax.experimental.pallas import tpu_sc as plsc

F32 = jnp.float32
BF16 = jnp.bfloat16
ET = 331776
TCB = 4096
GRID = ET // TCB
NW = 32
B2 = 96
I2 = ET // NW // B2
B6 = 128
I6 = ET // 16 // B6
NDUMMY = 240


def _k1_body(x_ref, lw_ref, sw_ref, dw_ref, xl_ref, as_ref, ad_ref):
    xb = x_ref[...]
    xl_ref[...] = jnp.dot(xb, lw_ref[...], preferred_element_type=F32)
    as_ref[...] = jnp.dot(xb, sw_ref[...], preferred_element_type=F32)
    ad_ref[...] = jnp.dot(xb, dw_ref[...], preferred_element_type=F32)


def _sc_gather(tsrc, tdst, src3, dst3):
    ws = tsrc.shape[1]
    wd = tdst.shape[1]
    mesh = plsc.VectorSubcoreMesh(core_axis_name="c", subcore_axis_name="s")
    per_w = ET // NW

    @functools.partial(
        pl.kernel, mesh=mesh,
        out_type=[
            jax.ShapeDtypeStruct((ET, ws), jnp.int32),
            jax.ShapeDtypeStruct((ET, wd), jnp.int32),
        ],
        scratch_types=[
            pltpu.VMEM((I2, B2), jnp.int32),
            pltpu.VMEM((I2, B2), jnp.int32),
            pltpu.VMEM((B2, ws), jnp.int32),
            pltpu.VMEM((B2, ws), jnp.int32),
            pltpu.VMEM((B2, wd), jnp.int32),
            pltpu.VMEM((B2, wd), jnp.int32),
            pltpu.SemaphoreType.DMA,
            pltpu.SemaphoreType.DMA,
            pltpu.SemaphoreType.DMA,
            pltpu.SemaphoreType.DMA,
        ],
    )
    def k2(tsrc_h, tdst_h, src_h, dst_h, gs_h, gd_h,
           ixs, ixd, bsA, bsB, bdA, bdB, sgA, sgB, swA, swB):
        wid = lax.axis_index("s") * 2 + lax.axis_index("c")
        base = wid * per_w
        pltpu.sync_copy(src_h.at[wid], ixs)
        pltpu.sync_copy(dst_h.at[wid], ixd)

        def g_start(i, bs, bd, sg):
            pltpu.async_copy(tsrc_h.at[ixs.at[i]], bs, sg)
            pltpu.async_copy(tdst_h.at[ixd.at[i]], bd, sg)

        def g_wait(i, bs, bd, sg):
            pltpu.make_async_copy(tsrc_h.at[ixs.at[i]], bs, sg).wait()
            pltpu.make_async_copy(tdst_h.at[ixd.at[i]], bd, sg).wait()

        def w_start(i, bs, bd, sw):
            off = base + i * B2
            pltpu.async_copy(bs, gs_h.at[pl.ds(off, B2)], sw)
            pltpu.async_copy(bd, gd_h.at[pl.ds(off, B2)], sw)

        def w_wait(i, bs, bd, sw):
            off = base + i * B2
            pltpu.make_async_copy(bs, gs_h.at[pl.ds(off, B2)], sw).wait()
            pltpu.make_async_copy(bd, gd_h.at[pl.ds(off, B2)], sw).wait()

        g_start(0, bsA, bdA, sgA)

        @pl.loop(0, I2 // 2)
        def _(p):
            i0 = 2 * p
            i1 = i0 + 1
            g_wait(i0, bsA, bdA, sgA)
            w_start(i0, bsA, bdA, swA)

            @pl.when(p > 0)
            def _():
                w_wait(i0 - 1, bsB, bdB, swB)

            g_start(i1, bsB, bdB, sgB)
            w_wait(i0, bsA, bdA, swA)

            @pl.when(p < I2 // 2 - 1)
            def _():
                g_start(i0 + 2, bsA, bdA, sgA)

            g_wait(i1, bsB, bdB, sgB)
            w_start(i1, bsB, bdB, swB)

        w_wait(I2 - 1, bsB, bdB, swB)

    return k2(tsrc, tdst, src3, dst3)


def _hi(x):
    return lax.bitcast_convert_type(x & jnp.int32(-65536), F32)


def _lo(x):
    return lax.bitcast_convert_type(x << 16, F32)


def _pkhi(x):
    xi = lax.bitcast_convert_type(x, jnp.int32)
    r = xi + jnp.int32(0x7FFF) + (lax.shift_right_logical(xi, 16) & 1)
    return r & jnp.int32(-65536)


def _pk2(a, b):
    return _pkhi(a) | lax.shift_right_logical(_pkhi(b), 16)


def _k3_body(n, gs0_ref, gd0_ref, s2_ref, d2_ref, w1_ref, b1_ref,
             out_ref, acc_ref, cnt_ref):
    i = pl.program_id(0)

    @pl.when(i == 0)
    def _():
        acc_ref[...] = jnp.zeros_like(acc_ref)
        cnt_ref[0] = 0.0

    pdiff = _lo(gd0_ref[...]) - _lo(gs0_ref[...])
    h1 = jnp.dot(pdiff.astype(BF16), w1_ref[...],
                 preferred_element_type=F32) + b1_ref[...]
    acc_ref[0:1, :] += jnp.sum(h1, axis=0, keepdims=True)
    acc_ref[1:2, :] += jnp.sum(h1 * h1, axis=0, keepdims=True)
    cnt_ref[0] += jnp.sum((s2_ref[...] != d2_ref[...]).astype(F32))

    @pl.when(i == pl.num_programs(0) - 1)
    def _():
        dh = acc_ref.shape[1]
        cnt = cnt_ref[0] + float(n)
        out_ref[...] = jnp.concatenate(
            [acc_ref[0:2, :], jnp.full((1, dh), cnt, F32),
             jnp.zeros((5, dh), F32)], axis=0)


def _k4_body(gs0_ref, gd0_ref, st_ref, w1_ref, b1_ref,
             g_ref, be_ref, w2_ref, b2_ref, aw1_ref, ab1_ref,
             p1_ref, out_ref, acc_ref):
    i = pl.program_id(0)

    @pl.when(i == 0)
    def _():
        acc_ref[...] = jnp.zeros_like(acc_ref)

    cntv = st_ref[2:3, :]
    ndv = float(ET) - cntv
    b1 = b1_ref[...]
    muv = (st_ref[0:1, :] - ndv * b1) / cntv
    msq = (st_ref[1:2, :] - ndv * b1 * b1) / cntv
    varv = msq - muv * muv
    s1 = g_ref[...] * jax.lax.rsqrt(varv + 1e-5)
    sh = be_ref[...] - muv * s1

    gs0 = gs0_ref[...]
    gd0 = gd0_ref[...]
    pdiff = _lo(gd0) - _lo(gs0)
    h1 = jnp.dot(pdiff.astype(BF16), w1_ref[...],
                 preferred_element_type=F32) + b1
    hbn = jnp.maximum(h1 * s1 + sh, 0.0)
    delta = jnp.dot(hbn.astype(BF16), w2_ref[...],
                    preferred_element_type=F32) + b2_ref[...]
    adf = _hi(gd0) - _hi(gs0)
    p1_ref[...] = _pk2(delta, adf)
    h1a = jnp.dot((adf + delta).astype(BF16), aw1_ref[...],
                  preferred_element_type=F32) + ab1_ref[...]
    acc_ref[0:1, :] += jnp.sum(h1a, axis=0, keepdims=True)
    acc_ref[1:2, :] += jnp.sum(h1a * h1a, axis=0, keepdims=True)

    @pl.when(i == pl.num_programs(0) - 1)
    def _():
        dh = acc_ref.shape[1]
        hbn0 = jnp.maximum(b1 * s1 + sh, 0.0)
        delta0 = jnp.dot(hbn0.astype(BF16), w2_ref[...],
                         preferred_element_type=F32) + b2_ref[...]
        h1a0 = jnp.dot(delta0.astype(BF16), aw1_ref[...],
                       preferred_element_type=F32) + ab1_ref[...]
        amu = (acc_ref[0:1, :] - ndv * h1a0) / cntv
        asq = (acc_ref[1:2, :] - ndv * h1a0 * h1a0) / cntv
        avar = asq - amu * amu
        out_ref[...] = jnp.concatenate(
            [amu, avar, jnp.zeros((6, dh), F32)], axis=0)


def _k5_body(p1_ref, ap_ref, ag_ref, abe_ref, aw1_ref, ab1_ref,
             aw2_ref, ab2_ref, u_ref, mx_ref, macc_ref):
    i = pl.program_id(0)

    @pl.when(i == 0)
    def _():
        macc_ref[...] = jnp.full_like(macc_ref, -3e38)

    s2 = ag_ref[...] * jax.lax.rsqrt(ap_ref[1:2, :] + 1e-5)
    sh2 = abe_ref[...] - ap_ref[0:1, :] * s2
    p1 = p1_ref[...]
    h1a = jnp.dot((_lo(p1) + _hi(p1)).astype(BF16), aw1_ref[...],
                  preferred_element_type=F32) + ab1_ref[...]
    ha = jnp.maximum(h1a * s2 + sh2, 0.0)
    u = jnp.dot(ha.astype(BF16), aw2_ref[...],
                preferred_element_type=F32) + ab2_ref[...]
    u_ref[...] = _pk2(u, _hi(p1))
    macc_ref[0:1, :] = jnp.maximum(macc_ref[0:1, :],
                                   jnp.max(u, axis=0, keepdims=True))

    @pl.when(i == pl.num_programs(0) - 1)
    def _():
        mx_ref[...] = jnp.broadcast_to(macc_ref[0:1, :], mx_ref.shape)


def _k5b_body(p2_ref, gs1_ref, mx_ref, e_ref, m_ref):
    p2 = p2_ref[...]
    e = jnp.exp(_hi(p2) - mx_ref[0:1, :])
    e_ref[...] = e
    m_ref[...] = e * (_hi(gs1_ref[...]) + _lo(p2))


def _sc_scatter(m, e, dst3, nacc, init=None):
    d = m.shape[1]
    mesh = plsc.VectorSubcoreMesh(core_axis_name="c", subcore_axis_name="s")
    i6 = dst3.shape[1]
    per_sub = i6 * B6
    rps = nacc // 16

    @functools.partial(
        pl.kernel, mesh=mesh,
        out_type=[
            jax.ShapeDtypeStruct((nacc, d), F32),
            jax.ShapeDtypeStruct((nacc, d), F32),
        ],
        scratch_types=[
            pltpu.VMEM((2, B6), jnp.int32),
            pltpu.VMEM((B6, d), F32),
            pltpu.VMEM((B6, d), F32),
            pltpu.VMEM_SHARED((nacc, d), F32),
            pltpu.SemaphoreType.DMA,
            pltpu.SemaphoreType.DMA,
        ],
    )
    def k6(m_h, e_h, dst_h, *rest):
        if init is None:
            accm_h, acce_h, idx, rowA, rowB, spacc, sA, sB = rest
        else:
            im_h, ie_h, accm_h, acce_h, idx, rowA, rowB, spacc, sA, sB = rest
        c = lax.axis_index("c")
        sid = lax.axis_index("s")

        if init is None:
            @pl.loop(0, B6)
            def _(r):
                @pl.loop(0, d // 16)
                def _(q):
                    rowA[r, pl.ds(q * 16, 16)] = jnp.zeros((16,), F32)

            @pl.loop(0, rps // B6)
            def _(k):
                pltpu.sync_copy(rowA, spacc.at[pl.ds(sid * rps + k * B6, B6)])
        else:
            @pl.when(c == 0)
            def _():
                pltpu.sync_copy(im_h.at[pl.ds(sid * rps, rps)],
                                spacc.at[pl.ds(sid * rps, rps)])

            @pl.when(c == 1)
            def _():
                pltpu.sync_copy(ie_h.at[pl.ds(sid * rps, rps)],
                                spacc.at[pl.ds(sid * rps, rps)])

        plsc.subcore_barrier()

        def run(arr_h, out_h):
            def l_start(i, buf, sem):
                off = sid * per_sub + i * B6
                pltpu.async_copy(arr_h.at[pl.ds(off, B6)], buf, sem)

            def l_wait(i, buf, sem):
                off = sid * per_sub + i * B6
                pltpu.make_async_copy(arr_h.at[pl.ds(off, B6)], buf,
                                      sem).wait()

            l_start(0, rowA, sA)

            @pl.loop(0, i6 // 2)
            def _(p):
                i0 = 2 * p
                i1 = i0 + 1
                pltpu.sync_copy(dst_h.at[sid, pl.ds(i0, 2)], idx)
                l_wait(i0, rowA, sA)
                l_start(i1, rowB, sB)
                pltpu.sync_copy(rowA, spacc.at[idx.at[0]], add=True)
                l_wait(i1, rowB, sB)

                @pl.when(p < i6 // 2 - 1)
                def _():
                    l_start(i0 + 2, rowA, sA)

                pltpu.sync_copy(rowB, spacc.at[idx.at[1]], add=True)

            plsc.subcore_barrier()
            pltpu.sync_copy(spacc.at[pl.ds(sid * rps, rps)],
                            out_h.at[pl.ds(sid * rps, rps)])

        @pl.when(c == 0)
        def _():
            run(m_h, accm_h)

        @pl.when(c == 1)
        def _():
            run(e_h, acce_h)

    if init is None:
        return k6(m, e, dst3)
    return k6(m, e, dst3, init[0], init[1])


def _k7_body(num_ref, den_ref, out_ref):
    out_ref[...] = num_ref[...] / (den_ref[...] + 1e-16)


def _espec(w, col=0):
    return pl.BlockSpec((TCB, w), lambda i, c=col: (i, c))


def _cspec(shape):
    return pl.BlockSpec(shape, lambda i: (0, 0))


def kernel(x, pos, edge_index, lin_W, src_W, dst_W, pos_W1, pos_b1, pos_g,
           pos_be, pos_W2, pos_b2, attn_W1, attn_b1, attn_g, attn_be,
           attn_W2, attn_b2):
    n, d = x.shape
    e = edge_index.shape[1]
    dh = pos_W1.shape[1]
    pad = ET - e - n

    src0 = edge_index[0].astype(jnp.int32)
    dst0 = edge_index[1].astype(jnp.int32)
    loops = jnp.arange(n, dtype=jnp.int32)
    padi = jnp.arange(pad, dtype=jnp.int32) % n
    srcp = jnp.concatenate([src0, loops, padi])
    dstp = jnp.concatenate([dst0, loops, padi])
    nacc = n + NDUMMY
    maskv = jnp.concatenate([src0 != dst0, jnp.ones((n,), bool),
                             jnp.zeros((pad,), bool)])
    dmy = n + (jnp.arange(ET, dtype=jnp.int32) % NDUMMY)
    dsts = jnp.where(maskv, dstp, dmy)
    posp = jnp.pad(pos.astype(F32), ((0, 0), (0, 128 - pos.shape[1])))
    w1p = jnp.pad(pos_W1, ((0, 128 - pos_W1.shape[0]), (0, 0))).astype(BF16)
    pw2b = pos_W2.astype(BF16)
    aw1b = attn_W1.astype(BF16)
    aw2b = attn_W2.astype(BF16)
    pb1 = pos_b1.reshape(1, dh)
    pg = pos_g.reshape(1, dh)
    pbe = pos_be.reshape(1, dh)
    pb2 = pos_b2.reshape(1, d)
    ab1 = attn_b1.reshape(1, dh)
    ag = attn_g.reshape(1, dh)
    abe = attn_be.reshape(1, dh)
    ab2 = attn_b2.reshape(1, d)
    s2d = srcp.reshape(ET // 512, 512)
    d2d = dstp.reshape(ET // 512, 512)

    xl, aS, aD = pl.pallas_call(
        _k1_body,
        out_shape=[jax.ShapeDtypeStruct((n, d), F32)] * 3,
    )(x, lin_W, src_W, dst_W)

    def pack2(A, B):
        au = lax.bitcast_convert_type(A.astype(BF16), jnp.uint16)
        bu = lax.bitcast_convert_type(B.astype(BF16), jnp.uint16)
        w = (au.astype(jnp.uint32) << 16) | bu.astype(jnp.uint32)
        return lax.bitcast_convert_type(w, jnp.int32)

    tsrc = jnp.concatenate(
        [pack2(aS, posp), pack2(xl, jnp.zeros((n, d), F32))], axis=1)
    tdst = pack2(aD, posp)
    src3 = srcp.reshape(NW, I2, B2)
    dst3 = dstp.reshape(NW, I2, B2)
    gs, gd = _sc_gather(tsrc, tdst, src3, dst3)

    st = pl.pallas_call(
        functools.partial(_k3_body, n),
        grid=(GRID,),
        in_specs=[_espec(d, 0), _espec(d, 0),
                  pl.BlockSpec((8, 512), lambda i: (i, 0)),
                  pl.BlockSpec((8, 512), lambda i: (i, 0)),
                  _cspec((128, dh)), _cspec((1, dh))],
        out_specs=_cspec((8, dh)),
        out_shape=jax.ShapeDtypeStruct((8, dh), F32),
        scratch_shapes=[pltpu.VMEM((8, dh), F32), pltpu.SMEM((1,), F32)],
    )(gs, gd, s2d, d2d, w1p, pb1)

    p1, ap = pl.pallas_call(
        _k4_body,
        grid=(GRID,),
        in_specs=[_espec(d, 0), _espec(d, 0),
                  _cspec((8, dh)), _cspec((128, dh)), _cspec((1, dh)),
                  _cspec((1, dh)), _cspec((1, dh)), _cspec((dh, d)),
                  _cspec((1, d)), _cspec((d, dh)), _cspec((1, dh))],
        out_specs=[_espec(d), _cspec((8, dh))],
        out_shape=[jax.ShapeDtypeStruct((ET, d), jnp.int32),
                   jax.ShapeDtypeStruct((8, dh), F32)],
        scratch_shapes=[pltpu.VMEM((8, dh), F32)],
    )(gs, gd, st, w1p, pb1, pg, pbe, pw2b, pb2, aw1b, ab1)

    u, mx = pl.pallas_call(
        _k5_body,
        grid=(GRID,),
        in_specs=[_espec(d), _cspec((8, dh)), _cspec((1, dh)),
                  _cspec((1, dh)), _cspec((d, dh)), _cspec((1, dh)),
                  _cspec((dh, d)), _cspec((1, d))],
        out_specs=[_espec(d), _cspec((8, d))],
        out_shape=[jax.ShapeDtypeStruct((ET, d), jnp.int32),
                   jax.ShapeDtypeStruct((8, d), F32)],
        scratch_shapes=[pltpu.VMEM((8, d), F32)],
    )(p1, ap, ag, abe, aw1b, ab1, aw2b, ab2)

    g1 = 39
    h1 = g1 * TCB

    def k5b(goff, nblk):
        return pl.pallas_call(
            _k5b_body,
            grid=(nblk,),
            in_specs=[
                pl.BlockSpec((TCB, d), lambda i: (i + goff, 0)),
                pl.BlockSpec((TCB, d), lambda i: (i + goff, 1)),
                _cspec((8, d)),
            ],
            out_specs=[_espec(d), _espec(d)],
            out_shape=[jax.ShapeDtypeStruct((nblk * TCB, d), F32),
                       jax.ShapeDtypeStruct((nblk * TCB, d), F32)],
        )(u, gs, mx)

    ev_a, mv_a = k5b(0, g1)
    ev_b, mv_b = k5b(g1, GRID - g1)

    d3a = dsts[:h1].reshape(16, h1 // 16 // B6, B6)
    d3b = dsts[h1:].reshape(16, (ET - h1) // 16 // B6, B6)
    pacc = _sc_scatter(mv_a, ev_a, d3a, nacc)
    accm, acce = _sc_scatter(mv_b, ev_b, d3b, nacc, init=pacc)

    out = pl.pallas_call(
        _k7_body,
        out_shape=jax.ShapeDtypeStruct((n, d), F32),
    )(accm[:n], acce[:n])
    return out


# speedup vs baseline: 1.0547x; 1.0224x over previous
import functools

import jax
import jax.numpy as jnp
from jax import lax
from jax.experimental import pallas as pl
from jax.experimental.pallas import tpu as pltpu
from jax.experimental.pallas import tpu_sc as plsc

F32 = jnp.float32
BF16 = jnp.bfloat16
ET = 331776
TCB = 4096
GRID = ET // TCB
NW = 32
B2 = 96
I2 = ET // NW // B2
B6 = 128
I6 = ET // 16 // B6
NDUMMY = 240


def _k1_body(x_ref, lw_ref, sw_ref, dw_ref, xl_ref, as_ref, ad_ref):
    xb = x_ref[...]
    xl_ref[...] = jnp.dot(xb, lw_ref[...], preferred_element_type=F32)
    as_ref[...] = jnp.dot(xb, sw_ref[...], preferred_element_type=F32)
    ad_ref[...] = jnp.dot(xb, dw_ref[...], preferred_element_type=F32)


def _sc_gather(tsrc, tdst, src3, dst3):
    ws = tsrc.shape[1]
    wd = tdst.shape[1]
    mesh = plsc.VectorSubcoreMesh(core_axis_name="c", subcore_axis_name="s")
    per_w = ET // NW

    @functools.partial(
        pl.kernel, mesh=mesh,
        out_type=[
            jax.ShapeDtypeStruct((ET, ws), jnp.int32),
            jax.ShapeDtypeStruct((ET, wd), jnp.int32),
        ],
        scratch_types=[
            pltpu.VMEM((I2, B2), jnp.int32),
            pltpu.VMEM((I2, B2), jnp.int32),
            pltpu.VMEM((B2, ws), jnp.int32),
            pltpu.VMEM((B2, ws), jnp.int32),
            pltpu.VMEM((B2, wd), jnp.int32),
            pltpu.VMEM((B2, wd), jnp.int32),
            pltpu.SemaphoreType.DMA,
            pltpu.SemaphoreType.DMA,
            pltpu.SemaphoreType.DMA,
            pltpu.SemaphoreType.DMA,
        ],
    )
    def k2(tsrc_h, tdst_h, src_h, dst_h, gs_h, gd_h,
           ixs, ixd, bsA, bsB, bdA, bdB, sgA, sgB, swA, swB):
        wid = lax.axis_index("s") * 2 + lax.axis_index("c")
        base = wid * per_w
        pltpu.sync_copy(src_h.at[wid], ixs)
        pltpu.sync_copy(dst_h.at[wid], ixd)

        def g_start(i, bs, bd, sg):
            pltpu.async_copy(tsrc_h.at[ixs.at[i]], bs, sg)
            pltpu.async_copy(tdst_h.at[ixd.at[i]], bd, sg)

        def g_wait(i, bs, bd, sg):
            pltpu.make_async_copy(tsrc_h.at[ixs.at[i]], bs, sg).wait()
            pltpu.make_async_copy(tdst_h.at[ixd.at[i]], bd, sg).wait()

        def w_start(i, bs, bd, sw):
            off = base + i * B2
            pltpu.async_copy(bs, gs_h.at[pl.ds(off, B2)], sw)
            pltpu.async_copy(bd, gd_h.at[pl.ds(off, B2)], sw)

        def w_wait(i, bs, bd, sw):
            off = base + i * B2
            pltpu.make_async_copy(bs, gs_h.at[pl.ds(off, B2)], sw).wait()
            pltpu.make_async_copy(bd, gd_h.at[pl.ds(off, B2)], sw).wait()

        g_start(0, bsA, bdA, sgA)

        @pl.loop(0, I2 // 2)
        def _(p):
            i0 = 2 * p
            i1 = i0 + 1
            g_wait(i0, bsA, bdA, sgA)
            w_start(i0, bsA, bdA, swA)

            @pl.when(p > 0)
            def _():
                w_wait(i0 - 1, bsB, bdB, swB)

            g_start(i1, bsB, bdB, sgB)
            w_wait(i0, bsA, bdA, swA)

            @pl.when(p < I2 // 2 - 1)
            def _():
                g_start(i0 + 2, bsA, bdA, sgA)

            g_wait(i1, bsB, bdB, sgB)
            w_start(i1, bsB, bdB, swB)

        w_wait(I2 - 1, bsB, bdB, swB)

    return k2(tsrc, tdst, src3, dst3)


def _hi(x):
    return lax.bitcast_convert_type(x & jnp.int32(-65536), F32)


def _lo(x):
    return lax.bitcast_convert_type(x << 16, F32)


def _pkhi(x):
    xi = lax.bitcast_convert_type(x, jnp.int32)
    r = xi + jnp.int32(0x7FFF) + (lax.shift_right_logical(xi, 16) & 1)
    return r & jnp.int32(-65536)


def _pk2(a, b):
    return _pkhi(a) | lax.shift_right_logical(_pkhi(b), 16)


def _k3_body(n, gs0_ref, gd0_ref, s2_ref, d2_ref, w1_ref, b1_ref,
             out_ref, acc_ref, cnt_ref):
    i = pl.program_id(0)

    @pl.when(i == 0)
    def _():
        acc_ref[...] = jnp.zeros_like(acc_ref)
        cnt_ref[0] = 0.0

    pdiff = _lo(gd0_ref[...]) - _lo(gs0_ref[...])
    h1 = jnp.dot(pdiff.astype(BF16), w1_ref[...],
                 preferred_element_type=F32) + b1_ref[...]
    acc_ref[0:1, :] += jnp.sum(h1, axis=0, keepdims=True)
    acc_ref[1:2, :] += jnp.sum(h1 * h1, axis=0, keepdims=True)
    cnt_ref[0] += jnp.sum((s2_ref[...] != d2_ref[...]).astype(F32))

    @pl.when(i == pl.num_programs(0) - 1)
    def _():
        dh = acc_ref.shape[1]
        cnt = cnt_ref[0] + float(n)
        out_ref[...] = jnp.concatenate(
            [acc_ref[0:2, :], jnp.full((1, dh), cnt, F32),
             jnp.zeros((5, dh), F32)], axis=0)


def _k4_body(gs0_ref, gd0_ref, st_ref, w1_ref, b1_ref,
             g_ref, be_ref, w2_ref, b2_ref, aw1_ref, ab1_ref,
             p1_ref, out_ref, acc_ref):
    i = pl.program_id(0)

    @pl.when(i == 0)
    def _():
        acc_ref[...] = jnp.zeros_like(acc_ref)

    cntv = st_ref[2:3, :]
    ndv = float(ET) - cntv
    b1 = b1_ref[...]
    muv = (st_ref[0:1, :] - ndv * b1) / cntv
    msq = (st_ref[1:2, :] - ndv * b1 * b1) / cntv
    varv = msq - muv * muv
    s1 = g_ref[...] * jax.lax.rsqrt(varv + 1e-5)
    sh = be_ref[...] - muv * s1

    gs0 = gs0_ref[...]
    gd0 = gd0_ref[...]
    pdiff = _lo(gd0) - _lo(gs0)
    h1 = jnp.dot(pdiff.astype(BF16), w1_ref[...],
                 preferred_element_type=F32) + b1
    hbn = jnp.maximum(h1 * s1 + sh, 0.0)
    delta = jnp.dot(hbn.astype(BF16), w2_ref[...],
                    preferred_element_type=F32) + b2_ref[...]
    adf = _hi(gd0) - _hi(gs0)
    p1_ref[...] = _pk2(delta, adf)
    h1a = jnp.dot((adf + delta).astype(BF16), aw1_ref[...],
                  preferred_element_type=F32) + ab1_ref[...]
    acc_ref[0:1, :] += jnp.sum(h1a, axis=0, keepdims=True)
    acc_ref[1:2, :] += jnp.sum(h1a * h1a, axis=0, keepdims=True)

    @pl.when(i == pl.num_programs(0) - 1)
    def _():
        dh = acc_ref.shape[1]
        hbn0 = jnp.maximum(b1 * s1 + sh, 0.0)
        delta0 = jnp.dot(hbn0.astype(BF16), w2_ref[...],
                         preferred_element_type=F32) + b2_ref[...]
        h1a0 = jnp.dot(delta0.astype(BF16), aw1_ref[...],
                       preferred_element_type=F32) + ab1_ref[...]
        amu = (acc_ref[0:1, :] - ndv * h1a0) / cntv
        asq = (acc_ref[1:2, :] - ndv * h1a0 * h1a0) / cntv
        avar = asq - amu * amu
        out_ref[...] = jnp.concatenate(
            [amu, avar, jnp.zeros((6, dh), F32)], axis=0)


def _k5_body(p1_ref, ap_ref, ag_ref, abe_ref, aw1_ref, ab1_ref,
             aw2_ref, ab2_ref, u_ref, mx_ref, macc_ref):
    i = pl.program_id(0)

    @pl.when(i == 0)
    def _():
        macc_ref[...] = jnp.full_like(macc_ref, -3e38)

    s2 = ag_ref[...] * jax.lax.rsqrt(ap_ref[1:2, :] + 1e-5)
    sh2 = abe_ref[...] - ap_ref[0:1, :] * s2
    p1 = p1_ref[...]
    h1a = jnp.dot((_lo(p1) + _hi(p1)).astype(BF16), aw1_ref[...],
                  preferred_element_type=F32) + ab1_ref[...]
    ha = jnp.maximum(h1a * s2 + sh2, 0.0)
    u = jnp.dot(ha.astype(BF16), aw2_ref[...],
                preferred_element_type=F32) + ab2_ref[...]
    u_ref[...] = _pk2(u, _hi(p1))
    macc_ref[0:1, :] = jnp.maximum(macc_ref[0:1, :],
                                   jnp.max(u, axis=0, keepdims=True))

    @pl.when(i == pl.num_programs(0) - 1)
    def _():
        mx_ref[...] = jnp.broadcast_to(macc_ref[0:1, :], mx_ref.shape)


def _k5b_body(p2_ref, gs1_ref, mx_ref, e_ref, m_ref):
    p2 = p2_ref[...]
    e = jnp.exp(_hi(p2) - mx_ref[0:1, :])
    e_ref[...] = e
    m_ref[...] = e * (_hi(gs1_ref[...]) + _lo(p2))


def _sc_scatter(m, e, dst3, nacc, init=None):
    d = m.shape[1]
    mesh = plsc.VectorSubcoreMesh(core_axis_name="c", subcore_axis_name="s")
    i6 = dst3.shape[1]
    per_sub = i6 * B6
    rps = nacc // 16

    @functools.partial(
        pl.kernel, mesh=mesh,
        out_type=[
            jax.ShapeDtypeStruct((nacc, d), F32),
            jax.ShapeDtypeStruct((nacc, d), F32),
        ],
        scratch_types=[
            pltpu.VMEM((2, B6), jnp.int32),
            pltpu.VMEM((B6, d), F32),
            pltpu.VMEM((B6, d), F32),
            pltpu.VMEM_SHARED((nacc, d), F32),
            pltpu.SemaphoreType.DMA,
            pltpu.SemaphoreType.DMA,
        ],
    )
    def k6(m_h, e_h, dst_h, *rest):
        if init is None:
            accm_h, acce_h, idx, rowA, rowB, spacc, sA, sB = rest
        else:
            im_h, ie_h, accm_h, acce_h, idx, rowA, rowB, spacc, sA, sB = rest
        c = lax.axis_index("c")
        sid = lax.axis_index("s")

        if init is None:
            @pl.loop(0, B6)
            def _(r):
                @pl.loop(0, d // 16)
                def _(q):
                    rowA[r, pl.ds(q * 16, 16)] = jnp.zeros((16,), F32)

            @pl.loop(0, rps // B6)
            def _(k):
                pltpu.sync_copy(rowA, spacc.at[pl.ds(sid * rps + k * B6, B6)])
        else:
            @pl.when(c == 0)
            def _():
                pltpu.sync_copy(im_h.at[pl.ds(sid * rps, rps)],
                                spacc.at[pl.ds(sid * rps, rps)])

            @pl.when(c == 1)
            def _():
                pltpu.sync_copy(ie_h.at[pl.ds(sid * rps, rps)],
                                spacc.at[pl.ds(sid * rps, rps)])

        plsc.subcore_barrier()

        def run(arr_h, out_h):
            def l_start(i, buf, sem):
                off = sid * per_sub + i * B6
                pltpu.async_copy(arr_h.at[pl.ds(off, B6)], buf, sem)

            def l_wait(i, buf, sem):
                off = sid * per_sub + i * B6
                pltpu.make_async_copy(arr_h.at[pl.ds(off, B6)], buf,
                                      sem).wait()

            l_start(0, rowA, sA)

            @pl.loop(0, i6 // 2)
            def _(p):
                i0 = 2 * p
                i1 = i0 + 1
                pltpu.sync_copy(dst_h.at[sid, pl.ds(i0, 2)], idx)
                l_wait(i0, rowA, sA)
                l_start(i1, rowB, sB)
                pltpu.sync_copy(rowA, spacc.at[idx.at[0]], add=True)
                l_wait(i1, rowB, sB)

                @pl.when(p < i6 // 2 - 1)
                def _():
                    l_start(i0 + 2, rowA, sA)

                pltpu.sync_copy(rowB, spacc.at[idx.at[1]], add=True)

            plsc.subcore_barrier()
            pltpu.sync_copy(spacc.at[pl.ds(sid * rps, rps)],
                            out_h.at[pl.ds(sid * rps, rps)])

        @pl.when(c == 0)
        def _():
            run(m_h, accm_h)

        @pl.when(c == 1)
        def _():
            run(e_h, acce_h)

    if init is None:
        return k6(m, e, dst3)
    return k6(m, e, dst3, init[0], init[1])


def _k7_body(num_ref, den_ref, out_ref):
    out_ref[...] = num_ref[...] / (den_ref[...] + 1e-16)


TCB2 = 6144
GRID2 = ET // TCB2


def _espec(w, col=0):
    return pl.BlockSpec((TCB, w), lambda i, c=col: (i, c))


def _espec2(w, col=0):
    return pl.BlockSpec((TCB2, w), lambda i, c=col: (i, c))


def _cspec(shape):
    return pl.BlockSpec(shape, lambda i: (0, 0))


def kernel(x, pos, edge_index, lin_W, src_W, dst_W, pos_W1, pos_b1, pos_g,
           pos_be, pos_W2, pos_b2, attn_W1, attn_b1, attn_g, attn_be,
           attn_W2, attn_b2):
    n, d = x.shape
    e = edge_index.shape[1]
    dh = pos_W1.shape[1]
    pad = ET - e - n

    src0 = edge_index[0].astype(jnp.int32)
    dst0 = edge_index[1].astype(jnp.int32)
    loops = jnp.arange(n, dtype=jnp.int32)
    padi = jnp.arange(pad, dtype=jnp.int32) % n
    srcp = jnp.concatenate([src0, loops, padi])
    dstp = jnp.concatenate([dst0, loops, padi])
    nacc = n + NDUMMY
    maskv = jnp.concatenate([src0 != dst0, jnp.ones((n,), bool),
                             jnp.zeros((pad,), bool)])
    dmy = n + (jnp.arange(ET, dtype=jnp.int32) % NDUMMY)
    dsts = jnp.where(maskv, dstp, dmy)
    posp = jnp.pad(pos.astype(F32), ((0, 0), (0, 128 - pos.shape[1])))
    w1p = jnp.pad(pos_W1, ((0, 128 - pos_W1.shape[0]), (0, 0))).astype(BF16)
    pw2b = pos_W2.astype(BF16)
    aw1b = attn_W1.astype(BF16)
    aw2b = attn_W2.astype(BF16)
    pb1 = pos_b1.reshape(1, dh)
    pg = pos_g.reshape(1, dh)
    pbe = pos_be.reshape(1, dh)
    pb2 = pos_b2.reshape(1, d)
    ab1 = attn_b1.reshape(1, dh)
    ag = attn_g.reshape(1, dh)
    abe = attn_be.reshape(1, dh)
    ab2 = attn_b2.reshape(1, d)
    s2d = srcp.reshape(ET // 512, 512)
    d2d = dstp.reshape(ET // 512, 512)

    xl, aS, aD = pl.pallas_call(
        _k1_body,
        out_shape=[jax.ShapeDtypeStruct((n, d), F32)] * 3,
    )(x, lin_W, src_W, dst_W)

    def pack2(A, B):
        au = lax.bitcast_convert_type(A.astype(BF16), jnp.uint16)
        bu = lax.bitcast_convert_type(B.astype(BF16), jnp.uint16)
        w = (au.astype(jnp.uint32) << 16) | bu.astype(jnp.uint32)
        return lax.bitcast_convert_type(w, jnp.int32)

    tsrc = jnp.concatenate(
        [pack2(aS, posp), pack2(xl, jnp.zeros((n, d), F32))], axis=1)
    tdst = pack2(aD, posp)
    src3 = srcp.reshape(NW, I2, B2)
    dst3 = dstp.reshape(NW, I2, B2)
    gs, gd = _sc_gather(tsrc, tdst, src3, dst3)

    st = pl.pallas_call(
        functools.partial(_k3_body, n),
        grid=(GRID,),
        in_specs=[_espec(d, 0), _espec(d, 0),
                  pl.BlockSpec((8, 512), lambda i: (i, 0)),
                  pl.BlockSpec((8, 512), lambda i: (i, 0)),
                  _cspec((128, dh)), _cspec((1, dh))],
        out_specs=_cspec((8, dh)),
        out_shape=jax.ShapeDtypeStruct((8, dh), F32),
        scratch_shapes=[pltpu.VMEM((8, dh), F32), pltpu.SMEM((1,), F32)],
    )(gs, gd, s2d, d2d, w1p, pb1)

    p1, ap = pl.pallas_call(
        _k4_body,
        grid=(GRID2,),
        in_specs=[_espec2(d, 0), _espec2(d, 0),
                  _cspec((8, dh)), _cspec((128, dh)), _cspec((1, dh)),
                  _cspec((1, dh)), _cspec((1, dh)), _cspec((dh, d)),
                  _cspec((1, d)), _cspec((d, dh)), _cspec((1, dh))],
        out_specs=[_espec2(d), _cspec((8, dh))],
        out_shape=[jax.ShapeDtypeStruct((ET, d), jnp.int32),
                   jax.ShapeDtypeStruct((8, dh), F32)],
        scratch_shapes=[pltpu.VMEM((8, dh), F32)],
    )(gs, gd, st, w1p, pb1, pg, pbe, pw2b, pb2, aw1b, ab1)

    u, mx = pl.pallas_call(
        _k5_body,
        grid=(GRID2,),
        in_specs=[_espec2(d), _cspec((8, dh)), _cspec((1, dh)),
                  _cspec((1, dh)), _cspec((d, dh)), _cspec((1, dh)),
                  _cspec((dh, d)), _cspec((1, d))],
        out_specs=[_espec2(d), _cspec((8, d))],
        out_shape=[jax.ShapeDtypeStruct((ET, d), jnp.int32),
                   jax.ShapeDtypeStruct((8, d), F32)],
        scratch_shapes=[pltpu.VMEM((8, d), F32)],
    )(p1, ap, ag, abe, aw1b, ab1, aw2b, ab2)

    g1 = 26
    h1 = g1 * TCB2

    def k5b(goff, nblk):
        return pl.pallas_call(
            _k5b_body,
            grid=(nblk,),
            in_specs=[
                pl.BlockSpec((TCB2, d), lambda i: (i + goff, 0)),
                pl.BlockSpec((TCB2, d), lambda i: (i + goff, 1)),
                _cspec((8, d)),
            ],
            out_specs=[_espec2(d), _espec2(d)],
            out_shape=[jax.ShapeDtypeStruct((nblk * TCB2, d), F32),
                       jax.ShapeDtypeStruct((nblk * TCB2, d), F32)],
        )(u, gs, mx)

    ev_a, mv_a = k5b(0, g1)
    ev_b, mv_b = k5b(g1, GRID2 - g1)

    d3a = dsts[:h1].reshape(16, h1 // 16 // B6, B6)
    d3b = dsts[h1:].reshape(16, (ET - h1) // 16 // B6, B6)
    pacc = _sc_scatter(mv_a, ev_a, d3a, nacc)
    accm, acce = _sc_scatter(mv_b, ev_b, d3b, nacc, init=pacc)

    out = pl.pallas_call(
        _k7_body,
        out_shape=jax.ShapeDtypeStruct((n, d), F32),
    )(accm[:n], acce[:n])
    return out
